# Initial kernel scaffold; baseline (speedup 1.0000x reference)
import jax
import jax.numpy as jnp
from jax.experimental import pallas as pl


def kernel(preds, labels):
    raise NotImplementedError("write your pallas kernel here")


# speedup vs baseline: 132.0239x; 132.0239x over previous
import functools

import jax
import jax.numpy as jnp
from jax.experimental import pallas as pl
from jax.experimental.pallas import tpu as pltpu

_B = 64
_T = 16384
_ROWS = 8
_NEG = float("-inf")


def _shift_right(x, k):
    r = pltpu.roll(x, k, axis=1)
    lane = jax.lax.broadcasted_iota(jnp.int32, x.shape, 1)
    return jnp.where(lane < k, _NEG, r)


def _shift_left(x, k):
    r = pltpu.roll(x, x.shape[1] - k, axis=1)
    lane = jax.lax.broadcasted_iota(jnp.int32, x.shape, 1)
    return jnp.where(lane >= x.shape[1] - k, _NEG, r)


def _peak_mask(v):
    f2 = jnp.maximum(v, _shift_right(v, 1))
    f4 = jnp.maximum(f2, _shift_right(f2, 2))
    f8 = jnp.maximum(f4, _shift_right(f4, 4))
    f12 = jnp.maximum(f8, _shift_right(f4, 8))
    leftmax = _shift_right(f12, 1)
    g2 = jnp.maximum(v, _shift_left(v, 1))
    g4 = jnp.maximum(g2, _shift_left(g2, 2))
    g8 = jnp.maximum(g4, _shift_left(g4, 4))
    g12 = jnp.maximum(g8, _shift_left(g4, 8))
    rightmax = _shift_left(g12, 1)
    return (v > leftmax) & (v >= rightmax)


def _onsets_kernel(p_ref, l_ref, out_ref):
    i = pl.program_id(0)

    @pl.when(i == 0)
    def _init():
        out_ref[0, 0] = jnp.int32(0)

    p = p_ref[...]
    l = l_ref[...]
    sp = _peak_mask(p).astype(jnp.int32)
    sl = _peak_mask(l).astype(jnp.int32)
    lane = jax.lax.broadcasted_iota(jnp.int32, sp.shape, 1)
    d = jnp.where(lane >= 1, sp - sl, 0)
    s = d
    k = 1
    while k < _T:
        r = pltpu.roll(s, k, axis=1)
        s = s + jnp.where(lane < k, 0, r)
        k *= 2
    total = s[:, -1:]
    block_sum = jnp.sum(jnp.abs(total - s))
    out_ref[0, 0] += block_sum


@jax.jit
def kernel(preds, labels):
    grid = _B // _ROWS
    acc = pl.pallas_call(
        _onsets_kernel,
        grid=(grid,),
        in_specs=[
            pl.BlockSpec((_ROWS, _T), lambda i: (i, 0)),
            pl.BlockSpec((_ROWS, _T), lambda i: (i, 0)),
        ],
        out_specs=pl.BlockSpec(memory_space=pltpu.SMEM),
        out_shape=jax.ShapeDtypeStruct((1, 1), jnp.int32),
    )(preds, labels)
    return acc[0, 0].astype(jnp.float32) / jnp.float32(_B * _T)
